# Initial kernel scaffold; baseline (speedup 1.0000x reference)
import jax
import jax.numpy as jnp
from jax.experimental import pallas as pl


def kernel(x, gate_w, gate_b, expert_w1, expert_b1, expert_w2, expert_b2):
    raise NotImplementedError("write your pallas kernel here")


# speedup vs baseline: 2.0971x; 2.0971x over previous
import functools

import jax
import jax.numpy as jnp
from jax.experimental import pallas as pl
from jax.experimental.pallas import tpu as pltpu

_D_MODEL = 768
_D_FF = 1536
_E = 8
_S = 2048


def _router_kernel(logits_ref, idx_ref, w8_ref, aux_ref):
    l = logits_ref[...]
    lanes = jax.lax.broadcasted_iota(jnp.int32, l.shape, 1)
    v1 = jnp.max(l, axis=1, keepdims=True)
    i1 = jnp.min(jnp.where(l == v1, lanes, _E), axis=1, keepdims=True)
    l_wo1 = jnp.where(lanes == i1, -jnp.inf, l)
    v2 = jnp.max(l_wo1, axis=1, keepdims=True)
    i2 = jnp.min(jnp.where(l_wo1 == v2, lanes, _E), axis=1, keepdims=True)
    kept = jnp.where(l >= v2, l, -1000000000.0)
    e = jnp.exp(kept - v1)
    p = e / jnp.sum(e, axis=1, keepdims=True)
    idx_ref[...] = jnp.concatenate([i1, i2], axis=1)
    w8_ref[...] = jnp.where((lanes == i1) | (lanes == i2), p, 0.0)
    usage = jnp.sum(p, axis=0, keepdims=True)
    imp = usage / jnp.sum(usage)
    mean = jnp.mean(imp)
    std = jnp.sqrt(jnp.mean((imp - mean) ** 2))
    aux_ref[...] = (std / (mean + 1e-10)).reshape(1, 1)


def _ffn_kernel(xn_ref, w1_ref, b1_ref, w2_ref, b2_ref, w8_ref, out_ref):
    e = pl.program_id(0)

    @pl.when(e == 0)
    def _():
        out_ref[...] = jnp.zeros_like(out_ref)

    lanes = jax.lax.broadcasted_iota(jnp.int32, (_S, _E), 1)
    wsel = jnp.sum(jnp.where(lanes == e, w8_ref[...], 0.0), axis=1,
                   keepdims=True)
    h = jnp.dot(xn_ref[...], w1_ref[0], preferred_element_type=jnp.float32)
    h = h + b1_ref[0]
    h = h * jax.nn.sigmoid(h)
    y = jnp.dot(h, w2_ref[0], preferred_element_type=jnp.float32)
    y = y + b2_ref[0]
    out_ref[...] += wsel * y


@functools.partial(jax.jit, static_argnums=())
def kernel(x, gate_w, gate_b, expert_w1, expert_b1, expert_w2, expert_b2):
    x = jnp.asarray(x, dtype=jnp.float32)
    scores = jnp.mean(x, axis=-1, keepdims=True)
    spiked_x = jnp.where(scores > 0.1, x, 0.0)
    xn = spiked_x / (jnp.sum(spiked_x, axis=-1, keepdims=True) + 1e-08)
    noise_key = jax.random.key(42)
    logits = jnp.einsum('bsd,de->bse', xn, gate_w) + gate_b
    logits = logits + jax.random.normal(
        jax.random.fold_in(noise_key, 1), logits.shape) * 0.01

    xn2 = xn.reshape(_S, _D_MODEL)
    logits2 = logits.reshape(_S, _E)

    idx, w8, aux = pl.pallas_call(
        _router_kernel,
        out_shape=(
            jax.ShapeDtypeStruct((_S, 2), jnp.int32),
            jax.ShapeDtypeStruct((_S, _E), jnp.float32),
            jax.ShapeDtypeStruct((1, 1), jnp.float32),
        ),
    )(logits2)

    out = pl.pallas_call(
        _ffn_kernel,
        grid=(_E,),
        in_specs=[
            pl.BlockSpec((_S, _D_MODEL), lambda e: (0, 0)),
            pl.BlockSpec((1, _D_MODEL, _D_FF), lambda e: (e, 0, 0)),
            pl.BlockSpec((1, 1, _D_FF), lambda e: (e, 0, 0)),
            pl.BlockSpec((1, _D_FF, _D_MODEL), lambda e: (e, 0, 0)),
            pl.BlockSpec((1, 1, _D_MODEL), lambda e: (e, 0, 0)),
            pl.BlockSpec((_S, _E), lambda e: (0, 0)),
        ],
        out_specs=pl.BlockSpec((_S, _D_MODEL), lambda e: (0, 0)),
        out_shape=jax.ShapeDtypeStruct((_S, _D_MODEL), jnp.float32),
    )(xn2, expert_w1, expert_b1.reshape(_E, 1, _D_FF), expert_w2,
      expert_b2.reshape(_E, 1, _D_MODEL), w8)

    return (out.reshape(x.shape), idx.reshape(1, _S, 2), aux[0, 0])
